# Initial kernel scaffold; baseline (speedup 1.0000x reference)
import jax
import jax.numpy as jnp
from jax.experimental import pallas as pl


def kernel(g1, g2, A1, A2):
    raise NotImplementedError("write your pallas kernel here")


# speedup vs baseline: 2.7120x; 2.7120x over previous
import functools
import math

import jax
import jax.numpy as jnp
from jax.experimental import pallas as pl
from jax.experimental.pallas import tpu as pltpu

_PI = math.pi

_ACOS_COEFFS = (
    -0.0012624911, 0.0066700901, -0.0170881256, 0.0308918810,
    -0.0501743046, 0.0889789874, -0.2145988016, 1.5707963050,
)


def _acos(x):
    ax = jnp.abs(x)
    p = jnp.float32(_ACOS_COEFFS[0])
    for c in _ACOS_COEFFS[1:]:
        p = p * ax + jnp.float32(c)
    r = jnp.sqrt(jnp.maximum(1.0 - ax, 0.0)) * p
    return jnp.where(x >= 0, r, _PI - r)


def _ag_kernel(a_ref, g_ref, o_ref):
    o_ref[...] = jax.lax.dot_general(
        a_ref[...], g_ref[...], (((1,), (0,)), ((), ())),
        preferred_element_type=jnp.float32)


def _theta_kernel(b1_ref, b2_ref, o_ref):
    b1 = b1_ref[...]
    b2 = b2_ref[...]
    d1 = jnp.sqrt(jnp.sum(b1 * b1, axis=1, keepdims=True))
    d2 = jnp.sqrt(jnp.sum(b2 * b2, axis=1, keepdims=True)).T
    agg = jax.lax.dot_general(
        b1, b2, (((1,), (1,)), ((), ())),
        preferred_element_type=jnp.float32)
    denom = d1 * d2
    s = jnp.clip(agg / denom, -0.9999, 0.9999)
    k1 = (s * (_PI - _acos(s)) + jnp.sqrt(1.0 - s * s)) / _PI
    degree = (_PI - _acos(k1)) / _PI
    o_ref[...] = agg * degree + k1 * denom


def _mm_kernel(x_ref, y_ref, o_ref, *, trans_y):
    dn = (((1,), (1 if trans_y else 0,)), ((), ()))
    o_ref[...] = jax.lax.dot_general(
        x_ref[...], y_ref[...], dn, preferred_element_type=jnp.float32)


def _matmul(x, y, trans_y, bm, bn):
    M, K = x.shape
    N = y.shape[0] if trans_y else y.shape[1]
    if trans_y:
        y_spec = pl.BlockSpec((bn, K), lambda m, n: (n, 0))
    else:
        y_spec = pl.BlockSpec((K, bn), lambda m, n: (0, n))
    return pl.pallas_call(
        functools.partial(_mm_kernel, trans_y=trans_y),
        grid=(M // bm, N // bn),
        in_specs=[pl.BlockSpec((bm, K), lambda m, n: (m, 0)), y_spec],
        out_specs=pl.BlockSpec((bm, bn), lambda m, n: (m, n)),
        out_shape=jax.ShapeDtypeStruct((M, N), jnp.float32),
        compiler_params=pltpu.CompilerParams(
            dimension_semantics=("parallel", "parallel")),
    )(x, y)


def _a_times_g(A, g, bm):
    M, K = A.shape
    D = g.shape[1]
    return pl.pallas_call(
        _ag_kernel,
        grid=(M // bm,),
        in_specs=[
            pl.BlockSpec((bm, K), lambda m: (m, 0)),
            pl.BlockSpec((K, D), lambda m: (0, 0)),
        ],
        out_specs=pl.BlockSpec((bm, D), lambda m: (m, 0)),
        out_shape=jax.ShapeDtypeStruct((M, D), jnp.float32),
        compiler_params=pltpu.CompilerParams(
            dimension_semantics=("parallel",)),
    )(A, g)


def _theta(B1, B2, bm, bn):
    M = B1.shape[0]
    N = B2.shape[0]
    D = B1.shape[1]
    return pl.pallas_call(
        _theta_kernel,
        grid=(M // bm, N // bn),
        in_specs=[
            pl.BlockSpec((bm, D), lambda m, n: (m, 0)),
            pl.BlockSpec((bn, D), lambda m, n: (n, 0)),
        ],
        out_specs=pl.BlockSpec((bm, bn), lambda m, n: (m, n)),
        out_shape=jax.ShapeDtypeStruct((M, N), jnp.float32),
        compiler_params=pltpu.CompilerParams(
            dimension_semantics=("parallel", "parallel")),
    )(B1, B2)


def kernel(g1, g2, A1, A2):
    B1 = _a_times_g(A1, g1, bm=512)
    B2 = _a_times_g(A2, g2, bm=512)
    theta = _theta(B1, B2, bm=512, bn=512)
    T = _matmul(A1, theta, trans_y=False, bm=512, bn=512)
    out = _matmul(T, A2, trans_y=True, bm=512, bn=512)
    return out


# speedup vs baseline: 2.9643x; 1.0930x over previous
import functools
import math

import jax
import jax.numpy as jnp
from jax.experimental import pallas as pl
from jax.experimental.pallas import tpu as pltpu

_PI = math.pi

_ACOS_COEFFS = (
    -0.0012624911, 0.0066700901, -0.0170881256, 0.0308918810,
    -0.0501743046, 0.0889789874, -0.2145988016, 1.5707963050,
)
_ACOS_COEFFS_PI = tuple(c / _PI for c in _ACOS_COEFFS)


def _acos(x):
    ax = jnp.abs(x)
    p = jnp.float32(_ACOS_COEFFS[0])
    for c in _ACOS_COEFFS[1:]:
        p = p * ax + jnp.float32(c)
    r = jnp.sqrt(jnp.maximum(1.0 - ax, 0.0)) * p
    return jnp.where(x >= 0, r, _PI - r)


def _ag_kernel(a_ref, g_ref, o_ref):
    o_ref[...] = jax.lax.dot_general(
        a_ref[...], g_ref[...], (((1,), (0,)), ((), ())),
        preferred_element_type=jnp.float32)


def _theta_kernel(b1_ref, b2_ref, o_ref):
    b1 = b1_ref[...]
    b2 = b2_ref[...]
    n1 = jnp.sum(b1 * b1, axis=1, keepdims=True)
    n2 = jnp.sum(b2 * b2, axis=1, keepdims=True)
    r1 = jax.lax.rsqrt(n1)
    r2t = jax.lax.rsqrt(n2).T
    d1 = n1 * r1
    d2t = (n2 * jax.lax.rsqrt(n2)).T
    agg = jax.lax.dot_general(
        b1, b2, (((1,), (1,)), ((), ())),
        preferred_element_type=jnp.float32)
    s = jnp.clip((agg * r1) * r2t, -0.9999, 0.9999)
    ax = jnp.abs(s)
    t = 1.0 - ax
    p = jnp.float32(_ACOS_COEFFS[0])
    for c in _ACOS_COEFFS[1:]:
        p = p * ax + jnp.float32(c)
    r = jnp.sqrt(t) * p
    acs = jnp.where(s >= 0, r, _PI - r)
    sq1 = jnp.sqrt(t * (1.0 + ax))
    k1 = (s * (_PI - acs) + sq1) * jnp.float32(1.0 / _PI)
    p2 = jnp.float32(_ACOS_COEFFS_PI[0])
    for c in _ACOS_COEFFS_PI[1:]:
        p2 = p2 * k1 + jnp.float32(c)
    degree = 1.0 - jnp.sqrt(1.0 - k1) * p2
    o_ref[...] = (agg * degree + (k1 * d1) * d2t).astype(o_ref.dtype)


def _mm_kernel(x_ref, y_ref, o_ref, *, trans_y):
    dn = (((1,), (1 if trans_y else 0,)), ((), ()))
    o_ref[...] = jax.lax.dot_general(
        x_ref[...], y_ref[...], dn,
        preferred_element_type=jnp.float32).astype(o_ref.dtype)


def _matmul(x, y, trans_y, bm, bn, out_dtype):
    M, K = x.shape
    N = y.shape[0] if trans_y else y.shape[1]
    if trans_y:
        y_spec = pl.BlockSpec((bn, K), lambda m, n: (n, 0))
    else:
        y_spec = pl.BlockSpec((K, bn), lambda m, n: (0, n))
    return pl.pallas_call(
        functools.partial(_mm_kernel, trans_y=trans_y),
        grid=(M // bm, N // bn),
        in_specs=[pl.BlockSpec((bm, K), lambda m, n: (m, 0)), y_spec],
        out_specs=pl.BlockSpec((bm, bn), lambda m, n: (m, n)),
        out_shape=jax.ShapeDtypeStruct((M, N), out_dtype),
        compiler_params=pltpu.CompilerParams(
            dimension_semantics=("parallel", "parallel")),
    )(x, y)


def _a_times_g(A, g, bm):
    M, K = A.shape
    D = g.shape[1]
    return pl.pallas_call(
        _ag_kernel,
        grid=(M // bm,),
        in_specs=[
            pl.BlockSpec((bm, K), lambda m: (m, 0)),
            pl.BlockSpec((K, D), lambda m: (0, 0)),
        ],
        out_specs=pl.BlockSpec((bm, D), lambda m: (m, 0)),
        out_shape=jax.ShapeDtypeStruct((M, D), jnp.float32),
        compiler_params=pltpu.CompilerParams(
            dimension_semantics=("parallel",)),
    )(A, g)


def _theta(B1, B2, bm, bn, out_dtype):
    M = B1.shape[0]
    N = B2.shape[0]
    D = B1.shape[1]
    return pl.pallas_call(
        _theta_kernel,
        grid=(M // bm, N // bn),
        in_specs=[
            pl.BlockSpec((bm, D), lambda m, n: (m, 0)),
            pl.BlockSpec((bn, D), lambda m, n: (n, 0)),
        ],
        out_specs=pl.BlockSpec((bm, bn), lambda m, n: (m, n)),
        out_shape=jax.ShapeDtypeStruct((M, N), out_dtype),
        compiler_params=pltpu.CompilerParams(
            dimension_semantics=("parallel", "parallel")),
    )(B1, B2)


def kernel(g1, g2, A1, A2):
    B1 = _a_times_g(A1, g1, bm=512)
    B2 = _a_times_g(A2, g2, bm=512)
    theta = _theta(B1, B2, bm=512, bn=512, out_dtype=jnp.bfloat16)
    A1b = A1.astype(jnp.bfloat16)
    A2b = A2.astype(jnp.bfloat16)
    T = _matmul(A1b, theta, trans_y=False, bm=512, bn=512,
                out_dtype=jnp.bfloat16)
    out = _matmul(T, A2b, trans_y=True, bm=512, bn=512,
                  out_dtype=jnp.float32)
    return out
